# baseline (speedup 1.0000x reference)
import jax
import jax.numpy as jnp
from jax.experimental import pallas as pl
from jax.experimental.pallas import tpu as pltpu

_BATCH = 1024
_N_CONT = 13
_N_CAT = 26
_VOCAB = 1000
_EMB = 50
_TOTAL_IN = _N_CONT + _N_CAT * _VOCAB
_TOTAL_OUT = _N_CONT + _N_CAT * _EMB
_TILE_B = 128


def _body(x_ref, emb_ref, o_ref):
    o_ref[:, :_N_CONT] = x_ref[:, :_N_CONT]
    for f in range(_N_CAT):
        x = x_ref[:, _N_CONT + f * _VOCAB : _N_CONT + (f + 1) * _VOCAB]
        o_ref[:, _N_CONT + f * _EMB : _N_CONT + (f + 1) * _EMB] = jnp.dot(
            x, emb_ref[f], preferred_element_type=jnp.float32
        )


def kernel(inputs, embeddings):
    return pl.pallas_call(
        _body,
        grid=(_BATCH // _TILE_B,),
        in_specs=[
            pl.BlockSpec((_TILE_B, _TOTAL_IN), lambda i: (i, 0)),
            pl.BlockSpec((_N_CAT, _VOCAB, _EMB), lambda i: (0, 0, 0)),
        ],
        out_specs=pl.BlockSpec((_TILE_B, _TOTAL_OUT), lambda i: (i, 0)),
        out_shape=jax.ShapeDtypeStruct((_BATCH, _TOTAL_OUT), jnp.float32),
    )(inputs, embeddings)
